# Initial kernel scaffold; baseline (speedup 1.0000x reference)
import jax
import jax.numpy as jnp
from jax.experimental import pallas as pl


def kernel(inputs, c_inputs, transition_matrices, adaptive_graph, Wq, Wk, Wv, gat1_W1, gat1_W2, gat2_W1, gat2_W2, wg_W, wg_b, wo_W, wo_b, ln_g, ln_b):
    raise NotImplementedError("write your pallas kernel here")


# speedup vs baseline: 1.1176x; 1.1176x over previous
import functools
import math

import jax
import jax.numpy as jnp
from jax.experimental import pallas as pl
from jax.experimental.pallas import tpu as pltpu

B, P, N, DM, H, DK, HID = 2, 12, 256, 128, 8, 16, 2
_SCALE = 1.0 / math.sqrt(DK)
_F32 = jnp.float32


def _dot_t(a, b):
    return jax.lax.dot_general(a, b, (((1,), (1,)), ((), ())),
                               preferred_element_type=_F32)


def _dot(a, b):
    return jax.lax.dot_general(a, b, (((1,), (0,)), ((), ())),
                               preferred_element_type=_F32)


def _body(x_ref, tm_ref, wq_ref, wk_ref, wv_ref, g1w1_ref, g1w2_ref,
          g2w1_ref, g2w2_ref, wg_ref, wgb_ref, wo_ref, wob_ref,
          lng_ref, lnb_ref, o_ref):
    x = x_ref[0]
    outs = []
    for i in range(HID):
        Q = _dot_t(x, wq_ref[i])
        K = _dot_t(x, wk_ref[i])
        V = _dot_t(x, wv_ref[i])
        tm = tm_ref[i]
        nz = tm != 0.0
        A_heads, S2_heads = [], []
        for h in range(H):
            Qh = Q[:, h * DK:(h + 1) * DK]
            Kh = K[:, h * DK:(h + 1) * DK]
            Vh = V[:, h * DK:(h + 1) * DK]
            S = _dot_t(Qh, Kh) * _SCALE
            S = jnp.where(nz, S, 0.0)
            mask = (S != 0.0).astype(_F32)
            m = jnp.max(S, axis=1, keepdims=True)
            e = jnp.exp(S - m) * mask
            alpha = e / (jnp.sum(e, axis=1, keepdims=True) + 1e-5)
            att = _dot(alpha * tm, Vh)
            A_heads.append(_dot(att, g1w1_ref[i, h]))
            S2_heads.append(jax.nn.relu(_dot(att, g1w2_ref[i, h])))
        A = jnp.stack(A_heads, axis=0)
        S2 = jnp.stack(S2_heads, axis=0)
        mx = jnp.max(S2, axis=0, keepdims=True)
        e2 = jnp.exp(S2 - mx)
        sm = e2 / jnp.sum(e2, axis=0, keepdims=True)
        outs.append(jnp.sum(A * sm, axis=0))

    A2 = [_dot(outs[g], g2w1_ref[g]) for g in range(HID)]
    S22 = [jax.nn.relu(_dot(outs[g], g2w2_ref[g])) for g in range(HID)]
    mx2 = jnp.maximum(S22[0], S22[1])
    e0 = jnp.exp(S22[0] - mx2)
    e1 = jnp.exp(S22[1] - mx2)
    den = e0 + e1
    out = A2[0] * (e0 / den) + A2[1] * (e1 / den)

    gg = _dot_t(x, wg_ref[...]) + wgb_ref[0]
    sw = gg * out
    sw = sw * jax.nn.sigmoid(sw)
    o2 = _dot_t(sw, wo_ref[...]) + wob_ref[0]
    y = x + o2
    mu = jnp.mean(y, axis=1, keepdims=True)
    var = jnp.mean((y - mu) ** 2, axis=1, keepdims=True)
    o_ref[0] = (y - mu) * jax.lax.rsqrt(var + 1e-5) * lng_ref[0] + lnb_ref[0]


def _full(shape):
    return pl.BlockSpec(shape, lambda i: (0,) * len(shape))


@functools.partial(jax.jit, static_argnames=())
def _run(x, tm, Wq, Wk, Wv, g1w1, g1w2, g2w1, g2w2, wg_W, wg_b,
         wo_W, wo_b, ln_g, ln_b):
    bp = B * P
    return pl.pallas_call(
        _body,
        grid=(bp,),
        in_specs=[
            pl.BlockSpec((1, N, DM), lambda i: (i, 0, 0)),
            _full((HID, N, N)),
            _full((HID, DM, DM)),
            _full((HID, DM, DM)),
            _full((HID, DM, DM)),
            _full((HID, H, DK, DM)),
            _full((HID, H, DK, DM)),
            _full((HID, DM, DM)),
            _full((HID, DM, DM)),
            _full((DM, DM)),
            _full((1, DM)),
            _full((DM, DM)),
            _full((1, DM)),
            _full((1, DM)),
            _full((1, DM)),
        ],
        out_specs=pl.BlockSpec((1, N, DM), lambda i: (i, 0, 0)),
        out_shape=jax.ShapeDtypeStruct((bp, N, DM), _F32),
        compiler_params=pltpu.CompilerParams(
            dimension_semantics=("parallel",)),
    )(x, tm, Wq, Wk, Wv, g1w1, g1w2, g2w1, g2w2, wg_W, wg_b,
      wo_W, wo_b, ln_g, ln_b)


def kernel(inputs, c_inputs, transition_matrices, adaptive_graph, Wq, Wk, Wv,
           gat1_W1, gat1_W2, gat2_W1, gat2_W2, wg_W, wg_b, wo_W, wo_b,
           ln_g, ln_b):
    x = inputs.reshape(B * P, N, DM)
    out = _run(x, transition_matrices, Wq, Wk, Wv, gat1_W1, gat1_W2,
               gat2_W1, gat2_W2, wg_W, wg_b.reshape(1, DM),
               wo_W, wo_b.reshape(1, DM), ln_g.reshape(1, DM),
               ln_b.reshape(1, DM))
    return out.reshape(B, P, N, DM)
